# Initial kernel scaffold; baseline (speedup 1.0000x reference)
import jax
import jax.numpy as jnp
from jax.experimental import pallas as pl


def kernel(x, edge_index, teamplate_node_mask, target_indices, edge_list, emb, conv_W, conv_b, conv_g, conv_be, lin_W, lin_b, lin_g, lin_be, out_W, out_b):
    raise NotImplementedError("write your pallas kernel here")


# speedup vs baseline: 9.8319x; 9.8319x over previous
import functools
import jax
import jax.numpy as jnp
from jax import lax
from jax.experimental import pallas as pl
from jax.experimental.pallas import tpu as pltpu, tpu_sc as plsc

N = 10000
D = 128
H = D // 2
OUT = 10
NC = 2
NS = 16
NW = NC * NS
N_PAD = 10240
RPT = N_PAD // NS
CHUNK = 128
CH = 160
GRP = 16
NG = CH // GRP
E_PAD = NS * CH * CHUNK
XB = 4
XCH = (N_PAD // NW) // XB

_mesh = plsc.VectorSubcoreMesh(core_axis_name="c", subcore_axis_name="s")
_sc_params = pltpu.CompilerParams(use_tc_tiling_on_sc=False)


@functools.partial(
    pl.kernel,
    mesh=_mesh,
    compiler_params=_sc_params,
    out_type=[
        jax.ShapeDtypeStruct((N_PAD, D), jnp.float32),
        jax.ShapeDtypeStruct((NC, N_PAD, 16), jnp.float32),
    ],
    scratch_types=[
        pltpu.VMEM((XB, XCH), jnp.int32),
        pltpu.VMEM((XCH, D), jnp.float32),
        pltpu.VMEM((CH // 2, CHUNK), jnp.int32),
        pltpu.VMEM((CHUNK, 16), jnp.float32),
        pltpu.VMEM_SHARED((N_PAD, 16), jnp.float32),
        pltpu.SemaphoreType.DMA,
    ],
)
def _sc_gather_deg(emb_hbm, x_hbm, dst_hbm, zeros16_hbm, ones16_hbm,
                   h0_hbm, degp_hbm,
                   xv, rowbuf, dstv, onesv, deg_acc, sem):
    c = lax.axis_index("c")
    s = lax.axis_index("s")
    wid = s * NC + c

    pltpu.sync_copy(x_hbm.at[wid], xv)
    for j in range(XB):
        pltpu.async_copy(emb_hbm.at[xv.at[j]], rowbuf, sem).wait()
        pltpu.sync_copy(rowbuf, h0_hbm.at[pl.ds(wid * (XB * XCH) + j * XCH, XCH)])

    pltpu.sync_copy(zeros16_hbm.at[pl.ds(s * RPT, RPT)],
                    deg_acc.at[pl.ds(s * RPT, RPT)])
    pltpu.sync_copy(ones16_hbm, onesv)
    pltpu.sync_copy(dst_hbm.at[s, pl.ds(c * (CH // 2), CH // 2)], dstv)
    plsc.subcore_barrier()

    def body(j, carry):
        pltpu.sync_copy(onesv, deg_acc.at[dstv.at[j]], add=True)
        return carry

    lax.fori_loop(0, CH // 2, body, 0)
    plsc.subcore_barrier()
    pltpu.sync_copy(deg_acc.at[pl.ds(s * RPT, RPT)],
                    degp_hbm.at[c, pl.ds(s * RPT, RPT)])


@functools.partial(
    pl.kernel,
    mesh=_mesh,
    compiler_params=_sc_params,
    out_type=jax.ShapeDtypeStruct((NC, N_PAD, H), jnp.float32),
    scratch_types=[
        pltpu.VMEM((GRP, CHUNK), jnp.int32),
        pltpu.VMEM((GRP, CHUNK), jnp.int32),
        pltpu.VMEM((CHUNK, H), jnp.float32),
        pltpu.VMEM((CHUNK, H), jnp.float32),
        pltpu.VMEM_SHARED((N_PAD, H), jnp.float32),
        pltpu.SemaphoreType.DMA,
        pltpu.SemaphoreType.DMA,
    ],
)
def _sc_edge_agg(hs_hbm, src_hbm, dst_hbm, zeros_hbm, p_hbm,
                 srcv, dstv, rows0, rows1, accum, sem0, sem1):
    c = lax.axis_index("c")
    s = lax.axis_index("s")

    pltpu.sync_copy(zeros_hbm.at[pl.ds(s * RPT, RPT)],
                    accum.at[pl.ds(s * RPT, RPT)])
    plsc.subcore_barrier()

    def group(g, carry):
        pltpu.sync_copy(src_hbm.at[s, pl.ds(g * GRP, GRP)], srcv)
        pltpu.sync_copy(dst_hbm.at[s, pl.ds(g * GRP, GRP)], dstv)

        pltpu.async_copy(hs_hbm.at[c].at[srcv.at[0]], rows0, sem0)

        def body(jj, carry2):
            j = jj * 2
            pltpu.async_copy(hs_hbm.at[c].at[srcv.at[j + 1]], rows1, sem1)
            pltpu.make_async_copy(hs_hbm.at[c].at[srcv.at[j]], rows0, sem0).wait()
            pltpu.sync_copy(rows0, accum.at[dstv.at[j]], add=True)

            @pl.when(jj + 1 < GRP // 2)
            def _():
                pltpu.async_copy(hs_hbm.at[c].at[srcv.at[j + 2]], rows0, sem0)

            pltpu.make_async_copy(hs_hbm.at[c].at[srcv.at[j + 1]], rows1,
                                  sem1).wait()
            pltpu.sync_copy(rows1, accum.at[dstv.at[j + 1]], add=True)
            return carry2

        lax.fori_loop(0, GRP // 2, body, 0)
        return carry

    lax.fori_loop(0, NG, group, 0)
    plsc.subcore_barrier()
    pltpu.sync_copy(accum.at[pl.ds(s * RPT, RPT)],
                    p_hbm.at[c, pl.ds(s * RPT, RPT)])


@functools.partial(
    pl.kernel,
    mesh=_mesh,
    compiler_params=_sc_params,
    out_type=jax.ShapeDtypeStruct((1024, D), jnp.float32),
    scratch_types=[
        pltpu.VMEM((32,), jnp.int32),
        pltpu.VMEM((32, D), jnp.float32),
        pltpu.SemaphoreType.DMA,
    ],
)
def _sc_target_gather(h_hbm, ti_hbm, out_hbm, tiv, rows, sem):
    c = lax.axis_index("c")
    s = lax.axis_index("s")
    wid = s * NC + c
    pltpu.sync_copy(ti_hbm.at[wid], tiv)
    pltpu.async_copy(h_hbm.at[tiv], rows, sem).wait()
    pltpu.sync_copy(rows, out_hbm.at[pl.ds(wid * 32, 32)])


_BLK = 1024
_GRID = N_PAD // _BLK

_half_spec = pl.BlockSpec((NC, _BLK, H), lambda i: (0, i, 0))
_full_spec = pl.BlockSpec((_BLK, D), lambda i: (i, 0))
_dinv_spec = pl.BlockSpec((_BLK, 1), lambda i: (i, 0))
_w_spec = pl.BlockSpec((D, D), lambda i: (0, 0))
_v_spec = pl.BlockSpec((1, D), lambda i: (0, 0))


def _tc_first_body(h0_ref, w_ref, d0_ref, d1_ref, hs_ref, dinv_ref):
    pid = pl.program_id(0)
    deg = d0_ref[:, 0:1] + d1_ref[:, 0:1] + 1.0
    rows = lax.broadcasted_iota(jnp.int32, (_BLK, 1), 0) + pid * _BLK
    dinv = jnp.where(rows < N, lax.rsqrt(deg), 0.0)
    hw = jnp.dot(h0_ref[...], w_ref[...], preferred_element_type=jnp.float32)
    hs = hw * dinv
    hs_ref[0] = hs[:, :H]
    hs_ref[1] = hs[:, H:]
    dinv_ref[...] = dinv


def _tc_first(h0, w0, d0, d1):
    return pl.pallas_call(
        _tc_first_body,
        grid=(_GRID,),
        in_specs=[
            _full_spec,
            _w_spec,
            pl.BlockSpec((_BLK, 16), lambda i: (i, 0)),
            pl.BlockSpec((_BLK, 16), lambda i: (i, 0)),
        ],
        out_specs=[_half_spec, _dinv_spec],
        out_shape=[
            jax.ShapeDtypeStruct((NC, N_PAD, H), jnp.float32),
            jax.ShapeDtypeStruct((N_PAD, 1), jnp.float32),
        ],
    )(h0, w0, d0, d1)


def _halves_ln_relu(p_ref, hs_ref, dinv_ref, b_ref, g_ref, be_ref):
    dinv = dinv_ref[...]
    b = b_ref[...]
    g = g_ref[...]
    be = be_ref[...]
    z_lo = (p_ref[0] + hs_ref[0]) * dinv + b[:, :H]
    z_hi = (p_ref[1] + hs_ref[1]) * dinv + b[:, H:]
    mu = (jnp.sum(z_lo, axis=-1, keepdims=True)
          + jnp.sum(z_hi, axis=-1, keepdims=True)) * (1.0 / D)
    zc_lo = z_lo - mu
    zc_hi = z_hi - mu
    var = (jnp.sum(zc_lo * zc_lo, axis=-1, keepdims=True)
           + jnp.sum(zc_hi * zc_hi, axis=-1, keepdims=True)) * (1.0 / D)
    rs = lax.rsqrt(var + 1e-5)
    h_lo = jnp.maximum(zc_lo * rs * g[:, :H] + be[:, :H], 0.0)
    h_hi = jnp.maximum(zc_hi * rs * g[:, H:] + be[:, H:], 0.0)
    return h_lo, h_hi, dinv


def _tc_mid_body(p_ref, hs_ref, dinv_ref, b_ref, g_ref, be_ref, w_ref,
                 out_ref):
    h_lo, h_hi, dinv = _halves_ln_relu(p_ref, hs_ref, dinv_ref, b_ref, g_ref,
                                       be_ref)
    w = w_ref[...]
    hw = (jnp.dot(h_lo, w[:H, :], preferred_element_type=jnp.float32)
          + jnp.dot(h_hi, w[H:, :], preferred_element_type=jnp.float32))
    hs = hw * dinv
    out_ref[0] = hs[:, :H]
    out_ref[1] = hs[:, H:]


def _tc_mid(p, hs, dinv, b, g, be, w):
    return pl.pallas_call(
        _tc_mid_body,
        grid=(_GRID,),
        in_specs=[_half_spec, _half_spec, _dinv_spec, _v_spec, _v_spec,
                  _v_spec, _w_spec],
        out_specs=_half_spec,
        out_shape=jax.ShapeDtypeStruct((NC, N_PAD, H), jnp.float32),
    )(p, hs, dinv, b, g, be, w)


def _tc_last_body(p_ref, hs_ref, dinv_ref, b_ref, g_ref, be_ref, out_ref):
    h_lo, h_hi, _ = _halves_ln_relu(p_ref, hs_ref, dinv_ref, b_ref, g_ref,
                                    be_ref)
    out_ref[:, :H] = h_lo
    out_ref[:, H:] = h_hi


def _tc_last(p, hs, dinv, b, g, be):
    return pl.pallas_call(
        _tc_last_body,
        grid=(_GRID,),
        in_specs=[_half_spec, _half_spec, _dinv_spec, _v_spec, _v_spec,
                  _v_spec],
        out_specs=_full_spec,
        out_shape=jax.ShapeDtypeStruct((N_PAD, D), jnp.float32),
    )(p, hs, dinv, b, g, be)


def _tc_head_body(h_ref, w0_ref, b0_ref, g0_ref, be0_ref,
                  w1_ref, b1_ref, g1_ref, be1_ref, wo_ref, bo_ref, out_ref):
    def lin_ln_relu(h, w, b, g, be):
        z = jnp.dot(h, w, preferred_element_type=jnp.float32) + b
        mu = jnp.mean(z, axis=-1, keepdims=True)
        zc = z - mu
        var = jnp.mean(zc * zc, axis=-1, keepdims=True)
        zn = zc * lax.rsqrt(var + 1e-5) * g + be
        return jnp.maximum(zn, 0.0)

    h = h_ref[...]
    h = lin_ln_relu(h, w0_ref[...], b0_ref[...], g0_ref[...], be0_ref[...])
    h = lin_ln_relu(h, w1_ref[...], b1_ref[...], g1_ref[...], be1_ref[...])
    out_ref[...] = jnp.dot(h, wo_ref[...],
                           preferred_element_type=jnp.float32) + bo_ref[...]


def _tc_head(h, w0, b0, g0, be0, w1, b1, g1, be1, wo, bo):
    full = pl.BlockSpec((1024, D), lambda: (0, 0))
    wspec = pl.BlockSpec((D, D), lambda: (0, 0))
    vspec = pl.BlockSpec((1, D), lambda: (0, 0))
    return pl.pallas_call(
        _tc_head_body,
        in_specs=[full, wspec, vspec, vspec, vspec,
                  wspec, vspec, vspec, vspec, wspec, vspec],
        out_specs=pl.BlockSpec((1024, D), lambda: (0, 0)),
        out_shape=jax.ShapeDtypeStruct((1024, D), jnp.float32),
    )(h, w0, b0, g0, be0, w1, b1, g1, be1, wo, bo)


def kernel(x, edge_index, teamplate_node_mask, target_indices, edge_list,
           emb, conv_W, conv_b, conv_g, conv_be,
           lin_W, lin_b, lin_g, lin_be, out_W, out_b):
    f32 = jnp.float32
    i32 = jnp.int32

    x = jnp.ravel(x).astype(i32)
    x_pad = jnp.concatenate([x, jnp.zeros((N_PAD - N,), i32)]).reshape(NW, XB, XCH)

    edges = edge_list[0]
    epad = E_PAD - edges.shape[1]
    src_p = jnp.concatenate([edges[0].astype(i32), jnp.full((epad,), N, i32)])
    dst_p = jnp.concatenate([edges[1].astype(i32), jnp.full((epad,), N, i32)])
    src_p = src_p.reshape(NS, CH, CHUNK)
    dst_p = dst_p.reshape(NS, CH, CHUNK)

    ti = jnp.ravel(target_indices).astype(i32).reshape(NW, 32)

    zeros_h = jnp.zeros((N_PAD, H), f32)
    zeros16 = jnp.zeros((N_PAD, 16), f32)
    ones16 = jnp.ones((CHUNK, 16), f32)

    cb = conv_b.reshape(3, 1, D)
    cg = conv_g.reshape(3, 1, D)
    cbe = conv_be.reshape(3, 1, D)
    lb = lin_b.reshape(2, 1, D)
    lg = lin_g.reshape(2, 1, D)
    lbe = lin_be.reshape(2, 1, D)
    wo = jnp.zeros((D, D), f32).at[:, :OUT].set(out_W)
    bo = jnp.zeros((1, D), f32).at[0, :OUT].set(out_b)

    h0, degp = _sc_gather_deg(emb, x_pad, dst_p, zeros16, ones16)

    hs, dinv = _tc_first(h0, conv_W[0], degp[0], degp[1])

    for i in range(3):
        p = _sc_edge_agg(hs, src_p, dst_p, zeros_h)
        if i < 2:
            hs = _tc_mid(p, hs, dinv, cb[i], cg[i], cbe[i], conv_W[i + 1])
        else:
            h_fin = _tc_last(p, hs, dinv, cb[i], cg[i], cbe[i])

    hsel = _sc_target_gather(h_fin, ti)
    out = _tc_head(hsel, lin_W[0], lb[0], lg[0], lbe[0],
                   lin_W[1], lb[1], lg[1], lbe[1], wo, bo)
    return out[:, :OUT]


# speedup vs baseline: 10.4400x; 1.0619x over previous
import functools
import jax
import jax.numpy as jnp
from jax import lax
from jax.experimental import pallas as pl
from jax.experimental.pallas import tpu as pltpu, tpu_sc as plsc

N = 10000
D = 128
H = D // 2
OUT = 10
NC = 2
NS = 16
NW = NC * NS
N_PAD = 10240
RPT = N_PAD // NS
CHUNK = 128
CH = 160
GRP = 16
NG = CH // GRP
E_PAD = NS * CH * CHUNK
XB = 4
XCH = (N_PAD // NW) // XB

_mesh = plsc.VectorSubcoreMesh(core_axis_name="c", subcore_axis_name="s")
_sc_params = pltpu.CompilerParams(use_tc_tiling_on_sc=False)


@functools.partial(
    pl.kernel,
    mesh=_mesh,
    compiler_params=_sc_params,
    out_type=[
        jax.ShapeDtypeStruct((N_PAD, D), jnp.float32),
        jax.ShapeDtypeStruct((NC, N_PAD, 16), jnp.float32),
    ],
    scratch_types=[
        pltpu.VMEM((XB, XCH), jnp.int32),
        pltpu.VMEM((XCH, D), jnp.float32),
        pltpu.VMEM((CH // 2, CHUNK), jnp.int32),
        pltpu.VMEM((CHUNK, 16), jnp.float32),
        pltpu.VMEM_SHARED((N_PAD, 16), jnp.float32),
        pltpu.SemaphoreType.DMA,
    ],
)
def _sc_gather_deg(emb_hbm, x_hbm, dst_hbm, zeros16_hbm, ones16_hbm,
                   h0_hbm, degp_hbm,
                   xv, rowbuf, dstv, onesv, deg_acc, sem):
    c = lax.axis_index("c")
    s = lax.axis_index("s")
    wid = s * NC + c

    pltpu.sync_copy(x_hbm.at[wid], xv)
    for j in range(XB):
        pltpu.async_copy(emb_hbm.at[xv.at[j]], rowbuf, sem).wait()
        pltpu.sync_copy(rowbuf, h0_hbm.at[pl.ds(wid * (XB * XCH) + j * XCH, XCH)])

    pltpu.sync_copy(zeros16_hbm.at[pl.ds(s * RPT, RPT)],
                    deg_acc.at[pl.ds(s * RPT, RPT)])
    pltpu.sync_copy(ones16_hbm, onesv)
    pltpu.sync_copy(dst_hbm.at[s, pl.ds(c * (CH // 2), CH // 2)], dstv)
    plsc.subcore_barrier()

    def body(j, carry):
        pltpu.sync_copy(onesv, deg_acc.at[dstv.at[j]], add=True)
        return carry

    lax.fori_loop(0, CH // 2, body, 0)
    plsc.subcore_barrier()
    pltpu.sync_copy(deg_acc.at[pl.ds(s * RPT, RPT)],
                    degp_hbm.at[c, pl.ds(s * RPT, RPT)])


@functools.partial(
    pl.kernel,
    mesh=_mesh,
    compiler_params=_sc_params,
    out_type=jax.ShapeDtypeStruct((NC, N_PAD, H), jnp.float32),
    scratch_types=[
        pltpu.VMEM((CH, CHUNK), jnp.int32),
        pltpu.VMEM((CH, CHUNK), jnp.int32),
        pltpu.VMEM((CHUNK, H), jnp.float32),
        pltpu.VMEM((CHUNK, H), jnp.float32),
        pltpu.VMEM((CHUNK, H), jnp.float32),
        pltpu.VMEM((CHUNK, H), jnp.float32),
        pltpu.VMEM_SHARED((N_PAD, H), jnp.float32),
        pltpu.SemaphoreType.DMA,
        pltpu.SemaphoreType.DMA,
        pltpu.SemaphoreType.DMA,
        pltpu.SemaphoreType.DMA,
        pltpu.SemaphoreType.DMA,
        pltpu.SemaphoreType.DMA,
        pltpu.SemaphoreType.DMA,
        pltpu.SemaphoreType.DMA,
    ],
)
def _sc_edge_agg(hs_hbm, src_hbm, dst_hbm, zeros_hbm, p_hbm,
                 srcv, dstv, r0, r1, r2, r3, accum,
                 g0, g1, g2, g3, s0, s1, s2, s3):
    c = lax.axis_index("c")
    s = lax.axis_index("s")
    rows = (r0, r1, r2, r3)
    gsem = (g0, g1, g2, g3)
    ssem = (s0, s1, s2, s3)

    pltpu.sync_copy(zeros_hbm.at[pl.ds(s * RPT, RPT)],
                    accum.at[pl.ds(s * RPT, RPT)])
    pltpu.sync_copy(src_hbm.at[s], srcv)
    pltpu.sync_copy(dst_hbm.at[s], dstv)
    plsc.subcore_barrier()

    def _scatter(j, db2):
        pltpu.make_async_copy(hs_hbm.at[c].at[srcv.at[j]], rows[db2],
                              gsem[db2]).wait()
        pltpu.async_copy(rows[db2], accum.at[dstv.at[j]], ssem[db2], add=True)

    def quad(q, carry):
        for db in range(4):
            j = q * 4 + db

            @pl.when(q >= 1)
            def _(db=db, j=j):
                pltpu.make_async_copy(rows[db], accum.at[dstv.at[j - 4]],
                                      ssem[db]).wait()

            pltpu.async_copy(hs_hbm.at[c].at[srcv.at[j]], rows[db], gsem[db])

            db2 = (db + 2) % 4
            if db < 2:
                @pl.when(q >= 1)
                def _(j=j, db2=db2):
                    _scatter(j - 2, db2)
            else:
                _scatter(j - 2, db2)
        return carry

    lax.fori_loop(0, CH // 4, quad, 0)
    _scatter(CH - 2, (CH - 2) % 4)
    _scatter(CH - 1, (CH - 1) % 4)
    for db in range(4):
        j = CH - 4 + db
        pltpu.make_async_copy(rows[db], accum.at[dstv.at[j]], ssem[db]).wait()
    plsc.subcore_barrier()
    pltpu.sync_copy(accum.at[pl.ds(s * RPT, RPT)],
                    p_hbm.at[c, pl.ds(s * RPT, RPT)])


@functools.partial(
    pl.kernel,
    mesh=_mesh,
    compiler_params=_sc_params,
    out_type=jax.ShapeDtypeStruct((1024, D), jnp.float32),
    scratch_types=[
        pltpu.VMEM((32,), jnp.int32),
        pltpu.VMEM((32, D), jnp.float32),
        pltpu.SemaphoreType.DMA,
    ],
)
def _sc_target_gather(h_hbm, ti_hbm, out_hbm, tiv, rows, sem):
    c = lax.axis_index("c")
    s = lax.axis_index("s")
    wid = s * NC + c
    pltpu.sync_copy(ti_hbm.at[wid], tiv)
    pltpu.async_copy(h_hbm.at[tiv], rows, sem).wait()
    pltpu.sync_copy(rows, out_hbm.at[pl.ds(wid * 32, 32)])


_BLK = 1024
_GRID = N_PAD // _BLK

_half_spec = pl.BlockSpec((NC, _BLK, H), lambda i: (0, i, 0))
_full_spec = pl.BlockSpec((_BLK, D), lambda i: (i, 0))
_dinv_spec = pl.BlockSpec((_BLK, 1), lambda i: (i, 0))
_w_spec = pl.BlockSpec((D, D), lambda i: (0, 0))
_v_spec = pl.BlockSpec((1, D), lambda i: (0, 0))


def _tc_first_body(h0_ref, w_ref, d0_ref, d1_ref, hs_ref, dinv_ref):
    pid = pl.program_id(0)
    deg = d0_ref[:, 0:1] + d1_ref[:, 0:1] + 1.0
    rows = lax.broadcasted_iota(jnp.int32, (_BLK, 1), 0) + pid * _BLK
    dinv = jnp.where(rows < N, lax.rsqrt(deg), 0.0)
    hw = jnp.dot(h0_ref[...], w_ref[...], preferred_element_type=jnp.float32)
    hs = hw * dinv
    hs_ref[0] = hs[:, :H]
    hs_ref[1] = hs[:, H:]
    dinv_ref[...] = dinv


def _tc_first(h0, w0, d0, d1):
    return pl.pallas_call(
        _tc_first_body,
        grid=(_GRID,),
        in_specs=[
            _full_spec,
            _w_spec,
            pl.BlockSpec((_BLK, 16), lambda i: (i, 0)),
            pl.BlockSpec((_BLK, 16), lambda i: (i, 0)),
        ],
        out_specs=[_half_spec, _dinv_spec],
        out_shape=[
            jax.ShapeDtypeStruct((NC, N_PAD, H), jnp.float32),
            jax.ShapeDtypeStruct((N_PAD, 1), jnp.float32),
        ],
    )(h0, w0, d0, d1)


def _halves_ln_relu(p_ref, hs_ref, dinv_ref, b_ref, g_ref, be_ref):
    dinv = dinv_ref[...]
    b = b_ref[...]
    g = g_ref[...]
    be = be_ref[...]
    z_lo = (p_ref[0] + hs_ref[0]) * dinv + b[:, :H]
    z_hi = (p_ref[1] + hs_ref[1]) * dinv + b[:, H:]
    mu = (jnp.sum(z_lo, axis=-1, keepdims=True)
          + jnp.sum(z_hi, axis=-1, keepdims=True)) * (1.0 / D)
    zc_lo = z_lo - mu
    zc_hi = z_hi - mu
    var = (jnp.sum(zc_lo * zc_lo, axis=-1, keepdims=True)
           + jnp.sum(zc_hi * zc_hi, axis=-1, keepdims=True)) * (1.0 / D)
    rs = lax.rsqrt(var + 1e-5)
    h_lo = jnp.maximum(zc_lo * rs * g[:, :H] + be[:, :H], 0.0)
    h_hi = jnp.maximum(zc_hi * rs * g[:, H:] + be[:, H:], 0.0)
    return h_lo, h_hi, dinv


def _tc_mid_body(p_ref, hs_ref, dinv_ref, b_ref, g_ref, be_ref, w_ref,
                 out_ref):
    h_lo, h_hi, dinv = _halves_ln_relu(p_ref, hs_ref, dinv_ref, b_ref, g_ref,
                                       be_ref)
    w = w_ref[...]
    hw = (jnp.dot(h_lo, w[:H, :], preferred_element_type=jnp.float32)
          + jnp.dot(h_hi, w[H:, :], preferred_element_type=jnp.float32))
    hs = hw * dinv
    out_ref[0] = hs[:, :H]
    out_ref[1] = hs[:, H:]


def _tc_mid(p, hs, dinv, b, g, be, w):
    return pl.pallas_call(
        _tc_mid_body,
        grid=(_GRID,),
        in_specs=[_half_spec, _half_spec, _dinv_spec, _v_spec, _v_spec,
                  _v_spec, _w_spec],
        out_specs=_half_spec,
        out_shape=jax.ShapeDtypeStruct((NC, N_PAD, H), jnp.float32),
    )(p, hs, dinv, b, g, be, w)


def _tc_last_body(p_ref, hs_ref, dinv_ref, b_ref, g_ref, be_ref, out_ref):
    h_lo, h_hi, _ = _halves_ln_relu(p_ref, hs_ref, dinv_ref, b_ref, g_ref,
                                    be_ref)
    out_ref[:, :H] = h_lo
    out_ref[:, H:] = h_hi


def _tc_last(p, hs, dinv, b, g, be):
    return pl.pallas_call(
        _tc_last_body,
        grid=(_GRID,),
        in_specs=[_half_spec, _half_spec, _dinv_spec, _v_spec, _v_spec,
                  _v_spec],
        out_specs=_full_spec,
        out_shape=jax.ShapeDtypeStruct((N_PAD, D), jnp.float32),
    )(p, hs, dinv, b, g, be)


def _tc_head_body(h_ref, w0_ref, b0_ref, g0_ref, be0_ref,
                  w1_ref, b1_ref, g1_ref, be1_ref, wo_ref, bo_ref, out_ref):
    def lin_ln_relu(h, w, b, g, be):
        z = jnp.dot(h, w, preferred_element_type=jnp.float32) + b
        mu = jnp.mean(z, axis=-1, keepdims=True)
        zc = z - mu
        var = jnp.mean(zc * zc, axis=-1, keepdims=True)
        zn = zc * lax.rsqrt(var + 1e-5) * g + be
        return jnp.maximum(zn, 0.0)

    h = h_ref[...]
    h = lin_ln_relu(h, w0_ref[...], b0_ref[...], g0_ref[...], be0_ref[...])
    h = lin_ln_relu(h, w1_ref[...], b1_ref[...], g1_ref[...], be1_ref[...])
    out_ref[...] = jnp.dot(h, wo_ref[...],
                           preferred_element_type=jnp.float32) + bo_ref[...]


def _tc_head(h, w0, b0, g0, be0, w1, b1, g1, be1, wo, bo):
    full = pl.BlockSpec((1024, D), lambda: (0, 0))
    wspec = pl.BlockSpec((D, D), lambda: (0, 0))
    vspec = pl.BlockSpec((1, D), lambda: (0, 0))
    return pl.pallas_call(
        _tc_head_body,
        in_specs=[full, wspec, vspec, vspec, vspec,
                  wspec, vspec, vspec, vspec, wspec, vspec],
        out_specs=pl.BlockSpec((1024, D), lambda: (0, 0)),
        out_shape=jax.ShapeDtypeStruct((1024, D), jnp.float32),
    )(h, w0, b0, g0, be0, w1, b1, g1, be1, wo, bo)


def kernel(x, edge_index, teamplate_node_mask, target_indices, edge_list,
           emb, conv_W, conv_b, conv_g, conv_be,
           lin_W, lin_b, lin_g, lin_be, out_W, out_b):
    f32 = jnp.float32
    i32 = jnp.int32

    x = jnp.ravel(x).astype(i32)
    x_pad = jnp.concatenate([x, jnp.zeros((N_PAD - N,), i32)]).reshape(NW, XB, XCH)

    edges = edge_list[0]
    epad = E_PAD - edges.shape[1]
    src_p = jnp.concatenate([edges[0].astype(i32), jnp.full((epad,), N, i32)])
    dst_p = jnp.concatenate([edges[1].astype(i32), jnp.full((epad,), N, i32)])
    src_p = src_p.reshape(NS, CH, CHUNK)
    dst_p = dst_p.reshape(NS, CH, CHUNK)

    ti = jnp.ravel(target_indices).astype(i32).reshape(NW, 32)

    zeros_h = jnp.zeros((N_PAD, H), f32)
    zeros16 = jnp.zeros((N_PAD, 16), f32)
    ones16 = jnp.ones((CHUNK, 16), f32)

    cb = conv_b.reshape(3, 1, D)
    cg = conv_g.reshape(3, 1, D)
    cbe = conv_be.reshape(3, 1, D)
    lb = lin_b.reshape(2, 1, D)
    lg = lin_g.reshape(2, 1, D)
    lbe = lin_be.reshape(2, 1, D)
    wo = jnp.zeros((D, D), f32).at[:, :OUT].set(out_W)
    bo = jnp.zeros((1, D), f32).at[0, :OUT].set(out_b)

    h0, degp = _sc_gather_deg(emb, x_pad, dst_p, zeros16, ones16)

    hs, dinv = _tc_first(h0, conv_W[0], degp[0], degp[1])

    for i in range(3):
        p = _sc_edge_agg(hs, src_p, dst_p, zeros_h)
        if i < 2:
            hs = _tc_mid(p, hs, dinv, cb[i], cg[i], cbe[i], conv_W[i + 1])
        else:
            h_fin = _tc_last(p, hs, dinv, cb[i], cg[i], cbe[i])

    hsel = _sc_target_gather(h_fin, ti)
    out = _tc_head(hsel, lin_W[0], lb[0], lg[0], lbe[0],
                   lin_W[1], lb[1], lg[1], lbe[1], wo, bo)
    return out[:, :OUT]


# speedup vs baseline: 13.1107x; 1.2558x over previous
import functools
import jax
import jax.numpy as jnp
from jax import lax
from jax.experimental import pallas as pl
from jax.experimental.pallas import tpu as pltpu, tpu_sc as plsc

N = 10000
D = 128
H = D // 2
OUT = 10
NC = 2
NS = 16
NW = NC * NS
N_PAD = 10240
RPT = N_PAD // NS
CHUNK = 128
CH = 160
GRP = 16
NG = CH // GRP
E_PAD = NS * CH * CHUNK
XB = 4
XCH = (N_PAD // NW) // XB

_mesh = plsc.VectorSubcoreMesh(core_axis_name="c", subcore_axis_name="s")
_sc_params = pltpu.CompilerParams(use_tc_tiling_on_sc=False)
_sc_params_nl = pltpu.CompilerParams(use_tc_tiling_on_sc=False,
                                     needs_layout_passes=False)


@functools.partial(
    pl.kernel,
    mesh=_mesh,
    compiler_params=_sc_params,
    out_type=[
        jax.ShapeDtypeStruct((N_PAD, D), jnp.float32),
        jax.ShapeDtypeStruct((NC, N_PAD, 16), jnp.float32),
    ],
    scratch_types=[
        pltpu.VMEM((XB, XCH), jnp.int32),
        pltpu.VMEM((XCH, D), jnp.float32),
        pltpu.VMEM((CH // 2, CHUNK), jnp.int32),
        pltpu.VMEM((CHUNK, 16), jnp.float32),
        pltpu.VMEM_SHARED((N_PAD, 16), jnp.float32),
        pltpu.SemaphoreType.DMA,
    ],
)
def _sc_gather_deg(emb_hbm, x_hbm, dst_hbm, zeros16_hbm, ones16_hbm,
                   h0_hbm, degp_hbm,
                   xv, rowbuf, dstv, onesv, deg_acc, sem):
    c = lax.axis_index("c")
    s = lax.axis_index("s")
    wid = s * NC + c

    pltpu.sync_copy(x_hbm.at[wid], xv)
    for j in range(XB):
        pltpu.async_copy(emb_hbm.at[xv.at[j]], rowbuf, sem).wait()
        pltpu.sync_copy(rowbuf, h0_hbm.at[pl.ds(wid * (XB * XCH) + j * XCH, XCH)])

    pltpu.sync_copy(zeros16_hbm.at[pl.ds(s * RPT, RPT)],
                    deg_acc.at[pl.ds(s * RPT, RPT)])
    pltpu.sync_copy(ones16_hbm, onesv)
    pltpu.sync_copy(dst_hbm.at[s, pl.ds(c * (CH // 2), CH // 2)], dstv)
    plsc.subcore_barrier()

    def body(j, carry):
        pltpu.sync_copy(onesv, deg_acc.at[dstv.at[j]], add=True)
        return carry

    lax.fori_loop(0, CH // 2, body, 0)
    plsc.subcore_barrier()
    pltpu.sync_copy(deg_acc.at[pl.ds(s * RPT, RPT)],
                    degp_hbm.at[c, pl.ds(s * RPT, RPT)])


@functools.partial(
    pl.kernel,
    mesh=_mesh,
    compiler_params=_sc_params,
    out_type=jax.ShapeDtypeStruct((NC, N_PAD, H), jnp.float32),
    scratch_types=[
        pltpu.VMEM((CH, CHUNK), jnp.int32),
        pltpu.VMEM((CH, CHUNK), jnp.int32),
        pltpu.VMEM((CHUNK, H), jnp.float32),
        pltpu.VMEM((CHUNK, H), jnp.float32),
        pltpu.VMEM((CHUNK, H), jnp.float32),
        pltpu.VMEM((CHUNK, H), jnp.float32),
        pltpu.VMEM_SHARED((N_PAD, H), jnp.float32),
        pltpu.SemaphoreType.DMA,
        pltpu.SemaphoreType.DMA,
        pltpu.SemaphoreType.DMA,
        pltpu.SemaphoreType.DMA,
        pltpu.SemaphoreType.DMA,
        pltpu.SemaphoreType.DMA,
        pltpu.SemaphoreType.DMA,
        pltpu.SemaphoreType.DMA,
    ],
)
def _sc_edge_agg(hs_hbm, src_hbm, dst_hbm, zeros_hbm, p_hbm,
                 srcv, dstv, r0, r1, r2, r3, accum,
                 g0, g1, g2, g3, s0, s1, s2, s3):
    c = lax.axis_index("c")
    s = lax.axis_index("s")
    rows = (r0, r1, r2, r3)
    gsem = (g0, g1, g2, g3)
    ssem = (s0, s1, s2, s3)

    pltpu.sync_copy(zeros_hbm.at[pl.ds(s * RPT, RPT)],
                    accum.at[pl.ds(s * RPT, RPT)])
    pltpu.sync_copy(src_hbm.at[s], srcv)
    pltpu.sync_copy(dst_hbm.at[s], dstv)
    plsc.subcore_barrier()

    def _scatter(j, db2):
        pltpu.make_async_copy(hs_hbm.at[c].at[srcv.at[j]], rows[db2],
                              gsem[db2]).wait()
        pltpu.async_copy(rows[db2], accum.at[dstv.at[j]], ssem[db2], add=True)

    def quad(q, carry):
        for db in range(4):
            j = q * 4 + db

            @pl.when(q >= 1)
            def _(db=db, j=j):
                pltpu.make_async_copy(rows[db], accum.at[dstv.at[j - 4]],
                                      ssem[db]).wait()

            pltpu.async_copy(hs_hbm.at[c].at[srcv.at[j]], rows[db], gsem[db])

            db2 = (db + 2) % 4
            if db < 2:
                @pl.when(q >= 1)
                def _(j=j, db2=db2):
                    _scatter(j - 2, db2)
            else:
                _scatter(j - 2, db2)
        return carry

    lax.fori_loop(0, CH // 4, quad, 0)
    _scatter(CH - 2, (CH - 2) % 4)
    _scatter(CH - 1, (CH - 1) % 4)
    for db in range(4):
        j = CH - 4 + db
        pltpu.make_async_copy(rows[db], accum.at[dstv.at[j]], ssem[db]).wait()
    plsc.subcore_barrier()
    pltpu.sync_copy(accum.at[pl.ds(s * RPT, RPT)],
                    p_hbm.at[c, pl.ds(s * RPT, RPT)])


FCH = CH + 2
NSLOT = 1040
SPT = NSLOT // NS


@functools.partial(
    pl.kernel,
    mesh=_mesh,
    compiler_params=_sc_params_nl,
    out_type=[
        jax.ShapeDtypeStruct((NC, 1024, H), jnp.float32),
        jax.ShapeDtypeStruct((1024,), jnp.int32),
    ],
    scratch_types=[
        pltpu.VMEM((CH, CHUNK), jnp.int32),
        pltpu.VMEM((CH, CHUNK), jnp.int32),
        pltpu.VMEM((FCH, CHUNK), jnp.int32),
        pltpu.VMEM((FCH, CHUNK), jnp.int32),
        pltpu.VMEM((N_PAD,), jnp.int32),
        pltpu.VMEM((1024,), jnp.int32),
        pltpu.VMEM((64,), jnp.int32),
        pltpu.VMEM((CHUNK, H), jnp.float32),
        pltpu.VMEM((CHUNK, H), jnp.float32),
        pltpu.VMEM_SHARED((NSLOT, H), jnp.float32),
        pltpu.SemaphoreType.DMA,
        pltpu.SemaphoreType.DMA,
    ],
)
def _sc_edge_agg_targets(hs_hbm, src_hbm, dst_hbm, ti_hbm, zi32_hbm,
                         zeros_hbm, p_hbm, slots_hbm,
                         srcv, dstv, fsrc, fdst, mark, tiv, slotbuf,
                         rows0, rows1, accum, g0, g1):
    c = lax.axis_index("c")
    s = lax.axis_index("s")
    i32 = jnp.int32

    pltpu.sync_copy(zeros_hbm.at[pl.ds(s * SPT, SPT)],
                    accum.at[pl.ds(s * SPT, SPT)])
    pltpu.sync_copy(src_hbm.at[s], srcv)
    pltpu.sync_copy(dst_hbm.at[s], dstv)
    pltpu.sync_copy(zi32_hbm, mark)
    pltpu.sync_copy(ti_hbm, tiv)

    lane = lax.iota(i32, 16)
    for t in range(1024 // 16):
        tv = tiv[pl.ds(t * 16, 16)]
        plsc.store_scatter(mark, [tv], t * 16 + lane + 1)

    @pl.when(c == 0)
    def _():
        for t2 in range(4):
            tv = tiv[pl.ds(s * 64 + t2 * 16, 16)]
            slotbuf[pl.ds(t2 * 16, 16)] = plsc.load_gather(mark, [tv]) - 1
        pltpu.sync_copy(slotbuf, slots_hbm.at[pl.ds(s * 64, 64)])

    def filt(j, off):
        for k in range(CHUNK // 16):
            sv = srcv[j, pl.ds(k * 16, 16)]
            dv = dstv[j, pl.ds(k * 16, 16)]
            flags = plsc.load_gather(mark, [dv])
            msk = flags > 0
            mi = jnp.where(msk, 1, 0).astype(i32)
            pos = off + plsc.cumsum(mi) - 1
            row = lax.shift_right_logical(pos, 7)
            col = lax.bitwise_and(pos, 127)
            plsc.store_scatter(fsrc, [row, col], sv, mask=msk)
            plsc.store_scatter(fdst, [row, col], flags - 1, mask=msk)
            off = off + jnp.sum(mi)
        return off

    off = lax.fori_loop(0, CH, filt, jnp.asarray(0, i32))

    trash_s = jnp.full((16,), N, i32)
    trash_d = jnp.full((16,), 1024, i32)
    for t in range(256 // 16):
        pos = off + t * 16 + lane
        row = lax.shift_right_logical(pos, 7)
        col = lax.bitwise_and(pos, 127)
        plsc.store_scatter(fsrc, [row, col], trash_s)
        plsc.store_scatter(fdst, [row, col], trash_d)

    plsc.subcore_barrier()

    npair = jnp.maximum((off + 255) // 256, 1)

    pltpu.async_copy(hs_hbm.at[c].at[fsrc.at[0]], rows0, g0)

    def pair(jj, carry):
        j = jj * 2
        pltpu.async_copy(hs_hbm.at[c].at[fsrc.at[j + 1]], rows1, g1)
        pltpu.make_async_copy(hs_hbm.at[c].at[fsrc.at[j]], rows0, g0).wait()
        pltpu.sync_copy(rows0, accum.at[fdst.at[j]], add=True)

        @pl.when(jj + 1 < npair)
        def _():
            pltpu.async_copy(hs_hbm.at[c].at[fsrc.at[j + 2]], rows0, g0)

        pltpu.make_async_copy(hs_hbm.at[c].at[fsrc.at[j + 1]], rows1, g1).wait()
        pltpu.sync_copy(rows1, accum.at[fdst.at[j + 1]], add=True)
        return carry

    lax.fori_loop(0, npair, pair, 0)
    plsc.subcore_barrier()
    pltpu.sync_copy(accum.at[pl.ds(s * 64, 64)],
                    p_hbm.at[c, pl.ds(s * 64, 64)])


@functools.partial(
    pl.kernel,
    mesh=_mesh,
    compiler_params=_sc_params,
    out_type=[
        jax.ShapeDtypeStruct((NC, 1024, H), jnp.float32),
        jax.ShapeDtypeStruct((NC, 1024, H), jnp.float32),
        jax.ShapeDtypeStruct((NC, 1024, 16), jnp.float32),
    ],
    scratch_types=[
        pltpu.VMEM((64,), jnp.int32),
        pltpu.VMEM((64,), jnp.int32),
        pltpu.VMEM((64, H), jnp.float32),
        pltpu.VMEM((64, H), jnp.float32),
        pltpu.VMEM((64, 16), jnp.float32),
        pltpu.SemaphoreType.DMA,
    ],
)
def _sc_final_gather(p3c_hbm, hs_hbm, degp_hbm, ti_hbm, slots_hbm,
                     pt_hbm, hst_hbm, degt_hbm,
                     tiv, slotv, buf1, buf2, buf3, sem):
    c = lax.axis_index("c")
    s = lax.axis_index("s")
    pltpu.sync_copy(ti_hbm.at[s], tiv)
    pltpu.sync_copy(slots_hbm.at[pl.ds(s * 64, 64)], slotv)
    pltpu.async_copy(p3c_hbm.at[c].at[slotv], buf1, sem).wait()
    pltpu.sync_copy(buf1, pt_hbm.at[c, pl.ds(s * 64, 64)])
    pltpu.async_copy(hs_hbm.at[c].at[tiv], buf2, sem).wait()
    pltpu.sync_copy(buf2, hst_hbm.at[c, pl.ds(s * 64, 64)])
    pltpu.async_copy(degp_hbm.at[c].at[tiv], buf3, sem).wait()
    pltpu.sync_copy(buf3, degt_hbm.at[c, pl.ds(s * 64, 64)])


_BLK = 1024
_GRID = N_PAD // _BLK

_half_spec = pl.BlockSpec((NC, _BLK, H), lambda i: (0, i, 0))
_full_spec = pl.BlockSpec((_BLK, D), lambda i: (i, 0))
_dinv_spec = pl.BlockSpec((_BLK, 1), lambda i: (i, 0))
_w_spec = pl.BlockSpec((D, D), lambda i: (0, 0))
_v_spec = pl.BlockSpec((1, D), lambda i: (0, 0))


def _tc_first_body(h0_ref, w_ref, d0_ref, d1_ref, hs_ref, dinv_ref):
    pid = pl.program_id(0)
    deg = d0_ref[:, 0:1] + d1_ref[:, 0:1] + 1.0
    rows = lax.broadcasted_iota(jnp.int32, (_BLK, 1), 0) + pid * _BLK
    dinv = jnp.where(rows < N, lax.rsqrt(deg), 0.0)
    hw = jnp.dot(h0_ref[...], w_ref[...], preferred_element_type=jnp.float32)
    hs = hw * dinv
    hs_ref[0] = hs[:, :H]
    hs_ref[1] = hs[:, H:]
    dinv_ref[...] = dinv


def _tc_first(h0, w0, d0, d1):
    return pl.pallas_call(
        _tc_first_body,
        grid=(_GRID,),
        in_specs=[
            _full_spec,
            _w_spec,
            pl.BlockSpec((_BLK, 16), lambda i: (i, 0)),
            pl.BlockSpec((_BLK, 16), lambda i: (i, 0)),
        ],
        out_specs=[_half_spec, _dinv_spec],
        out_shape=[
            jax.ShapeDtypeStruct((NC, N_PAD, H), jnp.float32),
            jax.ShapeDtypeStruct((N_PAD, 1), jnp.float32),
        ],
    )(h0, w0, d0, d1)


def _halves_ln_relu(p_ref, hs_ref, dinv_ref, b_ref, g_ref, be_ref):
    dinv = dinv_ref[...]
    b = b_ref[...]
    g = g_ref[...]
    be = be_ref[...]
    z_lo = (p_ref[0] + hs_ref[0]) * dinv + b[:, :H]
    z_hi = (p_ref[1] + hs_ref[1]) * dinv + b[:, H:]
    mu = (jnp.sum(z_lo, axis=-1, keepdims=True)
          + jnp.sum(z_hi, axis=-1, keepdims=True)) * (1.0 / D)
    zc_lo = z_lo - mu
    zc_hi = z_hi - mu
    var = (jnp.sum(zc_lo * zc_lo, axis=-1, keepdims=True)
           + jnp.sum(zc_hi * zc_hi, axis=-1, keepdims=True)) * (1.0 / D)
    rs = lax.rsqrt(var + 1e-5)
    h_lo = jnp.maximum(zc_lo * rs * g[:, :H] + be[:, :H], 0.0)
    h_hi = jnp.maximum(zc_hi * rs * g[:, H:] + be[:, H:], 0.0)
    return h_lo, h_hi, dinv


def _tc_mid_body(p_ref, hs_ref, dinv_ref, b_ref, g_ref, be_ref, w_ref,
                 out_ref):
    h_lo, h_hi, dinv = _halves_ln_relu(p_ref, hs_ref, dinv_ref, b_ref, g_ref,
                                       be_ref)
    w = w_ref[...]
    hw = (jnp.dot(h_lo, w[:H, :], preferred_element_type=jnp.float32)
          + jnp.dot(h_hi, w[H:, :], preferred_element_type=jnp.float32))
    hs = hw * dinv
    out_ref[0] = hs[:, :H]
    out_ref[1] = hs[:, H:]


def _tc_mid(p, hs, dinv, b, g, be, w):
    return pl.pallas_call(
        _tc_mid_body,
        grid=(_GRID,),
        in_specs=[_half_spec, _half_spec, _dinv_spec, _v_spec, _v_spec,
                  _v_spec, _w_spec],
        out_specs=_half_spec,
        out_shape=jax.ShapeDtypeStruct((NC, N_PAD, H), jnp.float32),
    )(p, hs, dinv, b, g, be, w)


def _tc_final_body(pt_ref, hst_ref, degt_ref, b2_ref, g2_ref, be2_ref,
                   w0_ref, b0_ref, g0_ref, be0_ref,
                   w1_ref, b1_ref, g1_ref, be1_ref, wo_ref, bo_ref, out_ref):
    def lin_ln_relu(h, w, b, g, be):
        z = jnp.dot(h, w, preferred_element_type=jnp.float32) + b
        mu = jnp.mean(z, axis=-1, keepdims=True)
        zc = z - mu
        var = jnp.mean(zc * zc, axis=-1, keepdims=True)
        zn = zc * lax.rsqrt(var + 1e-5) * g + be
        return jnp.maximum(zn, 0.0)

    deg = degt_ref[0][:, 0:1] + degt_ref[1][:, 0:1] + 1.0
    dinv = lax.rsqrt(deg)
    b2 = b2_ref[...]
    g2 = g2_ref[...]
    be2 = be2_ref[...]
    z_lo = (pt_ref[0] + hst_ref[0]) * dinv + b2[:, :H]
    z_hi = (pt_ref[1] + hst_ref[1]) * dinv + b2[:, H:]
    mu = (jnp.sum(z_lo, axis=-1, keepdims=True)
          + jnp.sum(z_hi, axis=-1, keepdims=True)) * (1.0 / D)
    zc_lo = z_lo - mu
    zc_hi = z_hi - mu
    var = (jnp.sum(zc_lo * zc_lo, axis=-1, keepdims=True)
           + jnp.sum(zc_hi * zc_hi, axis=-1, keepdims=True)) * (1.0 / D)
    rs = lax.rsqrt(var + 1e-5)
    h_lo = jnp.maximum(zc_lo * rs * g2[:, :H] + be2[:, :H], 0.0)
    h_hi = jnp.maximum(zc_hi * rs * g2[:, H:] + be2[:, H:], 0.0)

    w0 = w0_ref[...]
    h = (jnp.dot(h_lo, w0[:H, :], preferred_element_type=jnp.float32)
         + jnp.dot(h_hi, w0[H:, :], preferred_element_type=jnp.float32)
         + b0_ref[...])
    mu = jnp.mean(h, axis=-1, keepdims=True)
    zc = h - mu
    var = jnp.mean(zc * zc, axis=-1, keepdims=True)
    h = jnp.maximum(zc * lax.rsqrt(var + 1e-5) * g0_ref[...] + be0_ref[...],
                    0.0)
    h = lin_ln_relu(h, w1_ref[...], b1_ref[...], g1_ref[...], be1_ref[...])
    out_ref[...] = jnp.dot(h, wo_ref[...],
                           preferred_element_type=jnp.float32) + bo_ref[...]


def _tc_final(pt, hst, degt, b2, g2, be2, w0, b0, g0, be0,
              w1, b1, g1, be1, wo, bo):
    hspec = pl.BlockSpec((NC, 1024, H), lambda: (0, 0, 0))
    wspec = pl.BlockSpec((D, D), lambda: (0, 0))
    vspec = pl.BlockSpec((1, D), lambda: (0, 0))
    return pl.pallas_call(
        _tc_final_body,
        in_specs=[hspec, hspec, pl.BlockSpec((NC, 1024, 16), lambda: (0, 0, 0)),
                  vspec, vspec, vspec,
                  wspec, vspec, vspec, vspec,
                  wspec, vspec, vspec, vspec, wspec, vspec],
        out_specs=pl.BlockSpec((1024, D), lambda: (0, 0)),
        out_shape=jax.ShapeDtypeStruct((1024, D), jnp.float32),
    )(pt, hst, degt, b2, g2, be2, w0, b0, g0, be0, w1, b1, g1, be1, wo, bo)


def kernel(x, edge_index, teamplate_node_mask, target_indices, edge_list,
           emb, conv_W, conv_b, conv_g, conv_be,
           lin_W, lin_b, lin_g, lin_be, out_W, out_b):
    f32 = jnp.float32
    i32 = jnp.int32

    x = jnp.ravel(x).astype(i32)
    x_pad = jnp.concatenate([x, jnp.zeros((N_PAD - N,), i32)]).reshape(NW, XB, XCH)

    edges = edge_list[0]
    epad = E_PAD - edges.shape[1]
    src_p = jnp.concatenate([edges[0].astype(i32), jnp.full((epad,), N, i32)])
    dst_p = jnp.concatenate([edges[1].astype(i32), jnp.full((epad,), N, i32)])
    src_p = src_p.reshape(NS, CH, CHUNK)
    dst_p = dst_p.reshape(NS, CH, CHUNK)

    ti_flat = jnp.ravel(target_indices).astype(i32)
    ti16 = ti_flat.reshape(NS, 64)
    zi32 = jnp.zeros((N_PAD,), i32)

    zeros_h = jnp.zeros((N_PAD, H), f32)
    zeros16 = jnp.zeros((N_PAD, 16), f32)
    ones16 = jnp.ones((CHUNK, 16), f32)

    cb = conv_b.reshape(3, 1, D)
    cg = conv_g.reshape(3, 1, D)
    cbe = conv_be.reshape(3, 1, D)
    lb = lin_b.reshape(2, 1, D)
    lg = lin_g.reshape(2, 1, D)
    lbe = lin_be.reshape(2, 1, D)
    wo = jnp.zeros((D, D), f32).at[:, :OUT].set(out_W)
    bo = jnp.zeros((1, D), f32).at[0, :OUT].set(out_b)

    h0, degp = _sc_gather_deg(emb, x_pad, dst_p, zeros16, ones16)

    hs, dinv = _tc_first(h0, conv_W[0], degp[0], degp[1])

    for i in range(2):
        p = _sc_edge_agg(hs, src_p, dst_p, zeros_h)
        hs = _tc_mid(p, hs, dinv, cb[i], cg[i], cbe[i], conv_W[i + 1])

    p3c, slots = _sc_edge_agg_targets(hs, src_p, dst_p, ti_flat, zi32, zeros_h)
    pt, hst, degt = _sc_final_gather(p3c, hs, degp, ti16, slots)
    out = _tc_final(pt, hst, degt, cb[2], cg[2], cbe[2],
                    lin_W[0], lb[0], lg[0], lbe[0],
                    lin_W[1], lb[1], lg[1], lbe[1], wo, bo)
    return out[:, :OUT]


# speedup vs baseline: 13.4433x; 1.0254x over previous
import functools
import jax
import jax.numpy as jnp
from jax import lax
from jax.experimental import pallas as pl
from jax.experimental.pallas import tpu as pltpu, tpu_sc as plsc

N = 10000
D = 128
H = D // 2
OUT = 10
NC = 2
NS = 16
NW = NC * NS
N_PAD = 10240
RPT = N_PAD // NS
CHUNK = 128
CH = 160
GRP = 16
NG = CH // GRP
E_PAD = NS * CH * CHUNK
XB = 4
XCH = (N_PAD // NW) // XB

_mesh = plsc.VectorSubcoreMesh(core_axis_name="c", subcore_axis_name="s")
_sc_params = pltpu.CompilerParams(use_tc_tiling_on_sc=False)
_sc_params_nl = pltpu.CompilerParams(use_tc_tiling_on_sc=False,
                                     needs_layout_passes=False)


@functools.partial(
    pl.kernel,
    mesh=_mesh,
    compiler_params=_sc_params,
    out_type=[
        jax.ShapeDtypeStruct((N_PAD, D), jnp.float32),
        jax.ShapeDtypeStruct((NC, N_PAD, 16), jnp.float32),
    ],
    scratch_types=[
        pltpu.VMEM((XB, XCH), jnp.int32),
        pltpu.VMEM((XCH, D), jnp.float32),
        pltpu.VMEM((XCH, D), jnp.float32),
        pltpu.VMEM((CH // 2, CHUNK), jnp.int32),
        pltpu.VMEM((CHUNK, 16), jnp.float32),
        pltpu.VMEM_SHARED((N_PAD, 16), jnp.float32),
        pltpu.SemaphoreType.DMA,
        pltpu.SemaphoreType.DMA,
        pltpu.SemaphoreType.DMA,
        pltpu.SemaphoreType.DMA,
    ],
)
def _sc_gather_deg(emb_hbm, x_hbm, dst_hbm, zeros16_hbm, ones16_hbm,
                   h0_hbm, degp_hbm,
                   xv, rb0, rb1, dstv, onesv, deg_acc, sg0, sg1, sw, sd):
    c = lax.axis_index("c")
    s = lax.axis_index("s")
    wid = s * NC + c

    rbs = (rb0, rb1)
    gse = (sg0, sg1)
    def _h0_slice(j):
        return h0_hbm.at[pl.ds(wid * (XB * XCH) + j * XCH, XCH)]

    pltpu.sync_copy(x_hbm.at[wid], xv)
    pltpu.async_copy(emb_hbm.at[xv.at[0]], rb0, sg0)
    for j in range(XB):
        b = j % 2
        if j + 1 < XB:
            if j >= 1:
                pltpu.make_async_copy(rbs[1 - b], _h0_slice(j - 1), sw).wait()
            pltpu.async_copy(emb_hbm.at[xv.at[j + 1]], rbs[1 - b], gse[1 - b])
        pltpu.make_async_copy(emb_hbm.at[xv.at[j]], rbs[b], gse[b]).wait()
        pltpu.async_copy(rbs[b], _h0_slice(j), sw)

    pltpu.sync_copy(zeros16_hbm.at[pl.ds(s * RPT, RPT)],
                    deg_acc.at[pl.ds(s * RPT, RPT)])
    pltpu.sync_copy(ones16_hbm, onesv)
    pltpu.sync_copy(dst_hbm.at[s, pl.ds(c * (CH // 2), CH // 2)], dstv)
    plsc.subcore_barrier()

    def body(g, carry):
        for k in range(4):
            pltpu.async_copy(onesv, deg_acc.at[dstv.at[g * 4 + k]], sd,
                             add=True)
        for k in range(4):
            pltpu.make_async_copy(onesv, deg_acc.at[dstv.at[g * 4 + k]],
                                  sd).wait()
        return carry

    lax.fori_loop(0, CH // 8, body, 0)
    for j in (XB - 2, XB - 1):
        pltpu.make_async_copy(rbs[j % 2], _h0_slice(j), sw).wait()
    plsc.subcore_barrier()
    pltpu.sync_copy(deg_acc.at[pl.ds(s * RPT, RPT)],
                    degp_hbm.at[c, pl.ds(s * RPT, RPT)])


@functools.partial(
    pl.kernel,
    mesh=_mesh,
    compiler_params=_sc_params,
    out_type=jax.ShapeDtypeStruct((NC, N_PAD, H), jnp.float32),
    scratch_types=[
        pltpu.VMEM((CH, CHUNK), jnp.int32),
        pltpu.VMEM((CH, CHUNK), jnp.int32),
        pltpu.VMEM((CHUNK, H), jnp.float32),
        pltpu.VMEM((CHUNK, H), jnp.float32),
        pltpu.VMEM((CHUNK, H), jnp.float32),
        pltpu.VMEM((CHUNK, H), jnp.float32),
        pltpu.VMEM_SHARED((N_PAD, H), jnp.float32),
        pltpu.SemaphoreType.DMA,
        pltpu.SemaphoreType.DMA,
        pltpu.SemaphoreType.DMA,
        pltpu.SemaphoreType.DMA,
        pltpu.SemaphoreType.DMA,
        pltpu.SemaphoreType.DMA,
        pltpu.SemaphoreType.DMA,
        pltpu.SemaphoreType.DMA,
    ],
)
def _sc_edge_agg(hs_hbm, src_hbm, dst_hbm, zeros_hbm, p_hbm,
                 srcv, dstv, r0, r1, r2, r3, accum,
                 g0, g1, g2, g3, s0, s1, s2, s3):
    c = lax.axis_index("c")
    s = lax.axis_index("s")
    rows = (r0, r1, r2, r3)
    gsem = (g0, g1, g2, g3)
    ssem = (s0, s1, s2, s3)

    pltpu.sync_copy(zeros_hbm.at[pl.ds(s * RPT, RPT)],
                    accum.at[pl.ds(s * RPT, RPT)])
    pltpu.sync_copy(src_hbm.at[s], srcv)
    pltpu.sync_copy(dst_hbm.at[s], dstv)
    plsc.subcore_barrier()

    def _scatter(j, db2):
        pltpu.make_async_copy(hs_hbm.at[c].at[srcv.at[j]], rows[db2],
                              gsem[db2]).wait()
        pltpu.async_copy(rows[db2], accum.at[dstv.at[j]], ssem[db2], add=True)

    def quad(q, carry):
        for db in range(4):
            j = q * 4 + db

            @pl.when(q >= 1)
            def _(db=db, j=j):
                pltpu.make_async_copy(rows[db], accum.at[dstv.at[j - 4]],
                                      ssem[db]).wait()

            pltpu.async_copy(hs_hbm.at[c].at[srcv.at[j]], rows[db], gsem[db])

            db2 = (db + 2) % 4
            if db < 2:
                @pl.when(q >= 1)
                def _(j=j, db2=db2):
                    _scatter(j - 2, db2)
            else:
                _scatter(j - 2, db2)
        return carry

    lax.fori_loop(0, CH // 4, quad, 0)
    _scatter(CH - 2, (CH - 2) % 4)
    _scatter(CH - 1, (CH - 1) % 4)
    for db in range(4):
        j = CH - 4 + db
        pltpu.make_async_copy(rows[db], accum.at[dstv.at[j]], ssem[db]).wait()
    plsc.subcore_barrier()
    pltpu.sync_copy(accum.at[pl.ds(s * RPT, RPT)],
                    p_hbm.at[c, pl.ds(s * RPT, RPT)])


FCH = CH + 2
NSLOT = 1040
SPT = NSLOT // NS


@functools.partial(
    pl.kernel,
    mesh=_mesh,
    compiler_params=_sc_params_nl,
    out_type=[
        jax.ShapeDtypeStruct((NC, 1024, H), jnp.float32),
        jax.ShapeDtypeStruct((NC, 1024, H), jnp.float32),
        jax.ShapeDtypeStruct((NC, 1024, 16), jnp.float32),
        jax.ShapeDtypeStruct((NC, 1024, H), jnp.float32),
    ],
    scratch_types=[
        pltpu.VMEM((CH, CHUNK), jnp.int32),
        pltpu.VMEM((CH, CHUNK), jnp.int32),
        pltpu.VMEM((FCH, CHUNK), jnp.int32),
        pltpu.VMEM((FCH, CHUNK), jnp.int32),
        pltpu.VMEM((N_PAD,), jnp.int32),
        pltpu.VMEM((1024,), jnp.int32),
        pltpu.VMEM((64,), jnp.int32),
        pltpu.VMEM((CHUNK, H), jnp.float32),
        pltpu.VMEM((CHUNK, H), jnp.float32),
        pltpu.VMEM((64, H), jnp.float32),
        pltpu.VMEM((64, 16), jnp.float32),
        pltpu.VMEM_SHARED((NSLOT, H), jnp.float32),
        pltpu.SemaphoreType.DMA,
        pltpu.SemaphoreType.DMA,
    ],
)
def _sc_edge_agg_targets(hs_hbm, src_hbm, dst_hbm, ti_hbm, zi32_hbm,
                         zeros_hbm, degp_hbm, pt_hbm, hst_hbm, degt_hbm,
                         p3c_hbm, srcv, dstv, fsrc, fdst, mark, tiv, slotbuf,
                         rows0, rows1, bufh, bufd, accum, g0, g1):
    c = lax.axis_index("c")
    s = lax.axis_index("s")
    i32 = jnp.int32

    pltpu.sync_copy(zeros_hbm.at[pl.ds(s * SPT, SPT)],
                    accum.at[pl.ds(s * SPT, SPT)])
    pltpu.sync_copy(src_hbm.at[s], srcv)
    pltpu.sync_copy(dst_hbm.at[s], dstv)
    pltpu.sync_copy(zi32_hbm, mark)
    pltpu.sync_copy(ti_hbm, tiv)

    lane = lax.iota(i32, 16)
    for t in range(1024 // 16):
        tv = tiv[pl.ds(t * 16, 16)]
        plsc.store_scatter(mark, [tv], t * 16 + lane + 1)

    for t2 in range(4):
        tv = tiv[pl.ds(s * 64 + t2 * 16, 16)]
        slotbuf[pl.ds(t2 * 16, 16)] = plsc.load_gather(mark, [tv]) - 1

    def filt(j, off):
        for k in range(CHUNK // 16):
            sv = srcv[j, pl.ds(k * 16, 16)]
            dv = dstv[j, pl.ds(k * 16, 16)]
            flags = plsc.load_gather(mark, [dv])
            msk = flags > 0
            mi = jnp.where(msk, 1, 0).astype(i32)
            pos = off + plsc.cumsum(mi) - 1
            row = lax.shift_right_logical(pos, 7)
            col = lax.bitwise_and(pos, 127)
            plsc.store_scatter(fsrc, [row, col], sv, mask=msk)
            plsc.store_scatter(fdst, [row, col], flags - 1, mask=msk)
            off = off + jnp.sum(mi)
        return off

    off = lax.fori_loop(0, CH, filt, jnp.asarray(0, i32))

    trash_s = jnp.full((16,), N, i32)
    trash_d = jnp.full((16,), 1024, i32)
    for t in range(256 // 16):
        pos = off + t * 16 + lane
        row = lax.shift_right_logical(pos, 7)
        col = lax.bitwise_and(pos, 127)
        plsc.store_scatter(fsrc, [row, col], trash_s)
        plsc.store_scatter(fdst, [row, col], trash_d)

    plsc.subcore_barrier()

    npair = jnp.maximum((off + 255) // 256, 1)

    pltpu.async_copy(hs_hbm.at[c].at[fsrc.at[0]], rows0, g0)

    def pair(jj, carry):
        j = jj * 2
        pltpu.async_copy(hs_hbm.at[c].at[fsrc.at[j + 1]], rows1, g1)
        pltpu.make_async_copy(hs_hbm.at[c].at[fsrc.at[j]], rows0, g0).wait()
        pltpu.sync_copy(rows0, accum.at[fdst.at[j]], add=True)

        @pl.when(jj + 1 < npair)
        def _():
            pltpu.async_copy(hs_hbm.at[c].at[fsrc.at[j + 2]], rows0, g0)

        pltpu.make_async_copy(hs_hbm.at[c].at[fsrc.at[j + 1]], rows1, g1).wait()
        pltpu.sync_copy(rows1, accum.at[fdst.at[j + 1]], add=True)
        return carry

    lax.fori_loop(0, npair, pair, 0)
    plsc.subcore_barrier()

    pltpu.sync_copy(accum.at[pl.ds(s * 64, 64)],
                    p3c_hbm.at[c, pl.ds(s * 64, 64)])
    plsc.subcore_barrier()
    tslice = tiv.at[pl.ds(s * 64, 64)]
    pltpu.async_copy(p3c_hbm.at[c].at[slotbuf], bufh, g0).wait()
    pltpu.sync_copy(bufh, pt_hbm.at[c, pl.ds(s * 64, 64)])
    pltpu.async_copy(hs_hbm.at[c].at[tslice], bufh, g0).wait()
    pltpu.sync_copy(bufh, hst_hbm.at[c, pl.ds(s * 64, 64)])
    pltpu.async_copy(degp_hbm.at[c].at[tslice], bufd, g0).wait()
    pltpu.sync_copy(bufd, degt_hbm.at[c, pl.ds(s * 64, 64)])


_BLK = 1024
_GRID = N_PAD // _BLK

_half_spec = pl.BlockSpec((NC, _BLK, H), lambda i: (0, i, 0))
_full_spec = pl.BlockSpec((_BLK, D), lambda i: (i, 0))
_dinv_spec = pl.BlockSpec((_BLK, 1), lambda i: (i, 0))
_w_spec = pl.BlockSpec((D, D), lambda i: (0, 0))
_v_spec = pl.BlockSpec((1, D), lambda i: (0, 0))


def _tc_first_body(h0_ref, w_ref, d0_ref, d1_ref, hs_ref, dinv_ref):
    pid = pl.program_id(0)
    deg = d0_ref[:, 0:1] + d1_ref[:, 0:1] + 1.0
    rows = lax.broadcasted_iota(jnp.int32, (_BLK, 1), 0) + pid * _BLK
    dinv = jnp.where(rows < N, lax.rsqrt(deg), 0.0)
    hw = jnp.dot(h0_ref[...], w_ref[...], preferred_element_type=jnp.float32)
    hs = hw * dinv
    hs_ref[0] = hs[:, :H]
    hs_ref[1] = hs[:, H:]
    dinv_ref[...] = dinv


def _tc_first(h0, w0, d0, d1):
    return pl.pallas_call(
        _tc_first_body,
        grid=(_GRID,),
        in_specs=[
            _full_spec,
            _w_spec,
            pl.BlockSpec((_BLK, 16), lambda i: (i, 0)),
            pl.BlockSpec((_BLK, 16), lambda i: (i, 0)),
        ],
        out_specs=[_half_spec, _dinv_spec],
        out_shape=[
            jax.ShapeDtypeStruct((NC, N_PAD, H), jnp.float32),
            jax.ShapeDtypeStruct((N_PAD, 1), jnp.float32),
        ],
    )(h0, w0, d0, d1)


def _halves_ln_relu(p_ref, hs_ref, dinv_ref, b_ref, g_ref, be_ref):
    dinv = dinv_ref[...]
    b = b_ref[...]
    g = g_ref[...]
    be = be_ref[...]
    z_lo = (p_ref[0] + hs_ref[0]) * dinv + b[:, :H]
    z_hi = (p_ref[1] + hs_ref[1]) * dinv + b[:, H:]
    mu = (jnp.sum(z_lo, axis=-1, keepdims=True)
          + jnp.sum(z_hi, axis=-1, keepdims=True)) * (1.0 / D)
    zc_lo = z_lo - mu
    zc_hi = z_hi - mu
    var = (jnp.sum(zc_lo * zc_lo, axis=-1, keepdims=True)
           + jnp.sum(zc_hi * zc_hi, axis=-1, keepdims=True)) * (1.0 / D)
    rs = lax.rsqrt(var + 1e-5)
    h_lo = jnp.maximum(zc_lo * rs * g[:, :H] + be[:, :H], 0.0)
    h_hi = jnp.maximum(zc_hi * rs * g[:, H:] + be[:, H:], 0.0)
    return h_lo, h_hi, dinv


def _tc_mid_body(p_ref, hs_ref, dinv_ref, b_ref, g_ref, be_ref, w_ref,
                 out_ref):
    h_lo, h_hi, dinv = _halves_ln_relu(p_ref, hs_ref, dinv_ref, b_ref, g_ref,
                                       be_ref)
    w = w_ref[...]
    hw = (jnp.dot(h_lo, w[:H, :], preferred_element_type=jnp.float32)
          + jnp.dot(h_hi, w[H:, :], preferred_element_type=jnp.float32))
    hs = hw * dinv
    out_ref[0] = hs[:, :H]
    out_ref[1] = hs[:, H:]


def _tc_mid(p, hs, dinv, b, g, be, w):
    return pl.pallas_call(
        _tc_mid_body,
        grid=(_GRID,),
        in_specs=[_half_spec, _half_spec, _dinv_spec, _v_spec, _v_spec,
                  _v_spec, _w_spec],
        out_specs=_half_spec,
        out_shape=jax.ShapeDtypeStruct((NC, N_PAD, H), jnp.float32),
    )(p, hs, dinv, b, g, be, w)


def _tc_final_body(pt_ref, hst_ref, degt_ref, b2_ref, g2_ref, be2_ref,
                   w0_ref, b0_ref, g0_ref, be0_ref,
                   w1_ref, b1_ref, g1_ref, be1_ref, wo_ref, bo_ref, out_ref):
    def lin_ln_relu(h, w, b, g, be):
        z = jnp.dot(h, w, preferred_element_type=jnp.float32) + b
        mu = jnp.mean(z, axis=-1, keepdims=True)
        zc = z - mu
        var = jnp.mean(zc * zc, axis=-1, keepdims=True)
        zn = zc * lax.rsqrt(var + 1e-5) * g + be
        return jnp.maximum(zn, 0.0)

    deg = degt_ref[0][:, 0:1] + degt_ref[1][:, 0:1] + 1.0
    dinv = lax.rsqrt(deg)
    b2 = b2_ref[...]
    g2 = g2_ref[...]
    be2 = be2_ref[...]
    z_lo = (pt_ref[0] + hst_ref[0]) * dinv + b2[:, :H]
    z_hi = (pt_ref[1] + hst_ref[1]) * dinv + b2[:, H:]
    mu = (jnp.sum(z_lo, axis=-1, keepdims=True)
          + jnp.sum(z_hi, axis=-1, keepdims=True)) * (1.0 / D)
    zc_lo = z_lo - mu
    zc_hi = z_hi - mu
    var = (jnp.sum(zc_lo * zc_lo, axis=-1, keepdims=True)
           + jnp.sum(zc_hi * zc_hi, axis=-1, keepdims=True)) * (1.0 / D)
    rs = lax.rsqrt(var + 1e-5)
    h_lo = jnp.maximum(zc_lo * rs * g2[:, :H] + be2[:, :H], 0.0)
    h_hi = jnp.maximum(zc_hi * rs * g2[:, H:] + be2[:, H:], 0.0)

    w0 = w0_ref[...]
    h = (jnp.dot(h_lo, w0[:H, :], preferred_element_type=jnp.float32)
         + jnp.dot(h_hi, w0[H:, :], preferred_element_type=jnp.float32)
         + b0_ref[...])
    mu = jnp.mean(h, axis=-1, keepdims=True)
    zc = h - mu
    var = jnp.mean(zc * zc, axis=-1, keepdims=True)
    h = jnp.maximum(zc * lax.rsqrt(var + 1e-5) * g0_ref[...] + be0_ref[...],
                    0.0)
    h = lin_ln_relu(h, w1_ref[...], b1_ref[...], g1_ref[...], be1_ref[...])
    out_ref[...] = jnp.dot(h, wo_ref[...],
                           preferred_element_type=jnp.float32) + bo_ref[...]


def _tc_final(pt, hst, degt, b2, g2, be2, w0, b0, g0, be0,
              w1, b1, g1, be1, wo, bo):
    hspec = pl.BlockSpec((NC, 1024, H), lambda: (0, 0, 0))
    wspec = pl.BlockSpec((D, D), lambda: (0, 0))
    vspec = pl.BlockSpec((1, D), lambda: (0, 0))
    return pl.pallas_call(
        _tc_final_body,
        in_specs=[hspec, hspec, pl.BlockSpec((NC, 1024, 16), lambda: (0, 0, 0)),
                  vspec, vspec, vspec,
                  wspec, vspec, vspec, vspec,
                  wspec, vspec, vspec, vspec, wspec, vspec],
        out_specs=pl.BlockSpec((1024, D), lambda: (0, 0)),
        out_shape=jax.ShapeDtypeStruct((1024, D), jnp.float32),
    )(pt, hst, degt, b2, g2, be2, w0, b0, g0, be0, w1, b1, g1, be1, wo, bo)


def kernel(x, edge_index, teamplate_node_mask, target_indices, edge_list,
           emb, conv_W, conv_b, conv_g, conv_be,
           lin_W, lin_b, lin_g, lin_be, out_W, out_b):
    f32 = jnp.float32
    i32 = jnp.int32

    x = jnp.ravel(x).astype(i32)
    x_pad = jnp.concatenate([x, jnp.zeros((N_PAD - N,), i32)]).reshape(NW, XB, XCH)

    edges = edge_list[0]
    epad = E_PAD - edges.shape[1]
    src_p = jnp.concatenate([edges[0].astype(i32), jnp.full((epad,), N, i32)])
    dst_p = jnp.concatenate([edges[1].astype(i32), jnp.full((epad,), N, i32)])
    src_p = src_p.reshape(NS, CH, CHUNK)
    dst_p = dst_p.reshape(NS, CH, CHUNK)

    ti_flat = jnp.ravel(target_indices).astype(i32)
    zi32 = jnp.zeros((N_PAD,), i32)

    zeros_h = jnp.zeros((N_PAD, H), f32)
    zeros16 = jnp.zeros((N_PAD, 16), f32)
    ones16 = jnp.ones((CHUNK, 16), f32)

    cb = conv_b.reshape(3, 1, D)
    cg = conv_g.reshape(3, 1, D)
    cbe = conv_be.reshape(3, 1, D)
    lb = lin_b.reshape(2, 1, D)
    lg = lin_g.reshape(2, 1, D)
    lbe = lin_be.reshape(2, 1, D)
    wo = jnp.zeros((D, D), f32).at[:, :OUT].set(out_W)
    bo = jnp.zeros((1, D), f32).at[0, :OUT].set(out_b)

    h0, degp = _sc_gather_deg(emb, x_pad, dst_p, zeros16, ones16)

    hs, dinv = _tc_first(h0, conv_W[0], degp[0], degp[1])

    for i in range(2):
        p = _sc_edge_agg(hs, src_p, dst_p, zeros_h)
        hs = _tc_mid(p, hs, dinv, cb[i], cg[i], cbe[i], conv_W[i + 1])

    pt, hst, degt, _ = _sc_edge_agg_targets(hs, src_p, dst_p, ti_flat, zi32,
                                            zeros_h, degp)
    out = _tc_final(pt, hst, degt, cb[2], cg[2], cbe[2],
                    lin_W[0], lb[0], lg[0], lbe[0],
                    lin_W[1], lb[1], lg[1], lbe[1], wo, bo)
    return out[:, :OUT]
